# baseline (speedup 1.0000x reference)
import jax
import jax.numpy as jnp
from jax import lax
from jax.experimental import pallas as pl
from jax.experimental.pallas import tpu as pltpu
from jax.experimental.pallas import tpu_sc as plsc

K = 64
D = 1024
N = 8192
NC = 2
NS = 16
L = 16
NG = 4
NT = 4
DG = 128
DH = NG * DG
TPG = N // NT
RPT = K // NT
CH = 128
NCH = TPG // CH


def _body(cent_hbm, st_hbm, keys_hbm, vals_hbm, asg_hbm, cnt0_hbm,
          outc_hbm, outs_hbm, outn_hbm,
          stage_k, stage_v, stage_c,
          acc_k, acc_v, cnt_acc,
          kbufa, vbufa, idxa, kbufb, vbufb, idxb,
          cbuf, sbuf, skbuf, svbuf, tbuf, nr, mr, ctmp, outcnt,
          sina, sinb):
    cid = lax.axis_index("c")
    sid = lax.axis_index("s")
    gl = sid // NT
    t = sid % NT
    gcol = cid * DH + gl * DG
    tok0 = t * TPG
    r0 = t * RPT

    zf16 = jnp.zeros((L,), jnp.float32)
    ones16 = jnp.ones((L,), jnp.float32)

    def _zf(r, _):
        for j in range(DG // L):
            acc_k[r, pl.ds(j * L, L)] = zf16
            acc_v[r, pl.ds(j * L, L)] = zf16
        cnt_acc[r, pl.ds(0, L)] = zf16
        return 0
    lax.fori_loop(0, K, _zf, 0)

    def _start_in(c, kb, vb, ib, sem):
        base = tok0 + c * CH
        pltpu.async_copy(asg_hbm.at[pl.ds(base, CH)], ib, sem)
        pltpu.async_copy(
            keys_hbm.at[pl.ds(base, CH), pl.ds(gcol, DG)], kb, sem)
        pltpu.async_copy(
            vals_hbm.at[pl.ds(base, CH), pl.ds(gcol, DG)], vb, sem)

    def _wait_in(kb, vb, ib, sem):
        pltpu.make_async_copy(asg_hbm.at[pl.ds(0, CH)], ib, sem).wait()
        pltpu.make_async_copy(
            keys_hbm.at[pl.ds(0, CH), pl.ds(0, DG)], kb, sem).wait()
        pltpu.make_async_copy(
            vals_hbm.at[pl.ds(0, CH), pl.ds(0, DG)], vb, sem).wait()

    def _compute(kb, vb, ib):
        @plsc.parallel_loop(0, CH // L, step=1, unroll=2)
        def _grp(q):
            iv = ib[pl.ds(q * L, L)]

            def _loads(tok):
                return (
                    [kb[tok, pl.ds(j * L, L)] for j in range(DG // L)],
                    [vb[tok, pl.ds(j * L, L)] for j in range(DG // L)],
                )

            cur = _loads(q * L)
            acur = iv[0]
            for tt in range(L):
                kcur, vcur = cur
                if tt + 1 < L:
                    anxt = iv[tt + 1]
                    tok = q * L + tt + 1
                    knxt, vnxt = [], []
                    for j in range(DG // L):
                        knxt.append(kb[tok, pl.ds(j * L, L)])
                        plsc.addupdate(acc_k.at[acur, pl.ds(j * L, L)],
                                       kcur[j])
                    for j in range(DG // L):
                        vnxt.append(vb[tok, pl.ds(j * L, L)])
                        plsc.addupdate(acc_v.at[acur, pl.ds(j * L, L)],
                                       vcur[j])
                    cur = (knxt, vnxt)
                    acur = anxt
                else:
                    for j in range(DG // L):
                        plsc.addupdate(acc_k.at[acur, pl.ds(j * L, L)],
                                       kcur[j])
                    for j in range(DG // L):
                        plsc.addupdate(acc_v.at[acur, pl.ds(j * L, L)],
                                       vcur[j])

        @pl.when(gl == 0)
        def _():
            @plsc.parallel_loop(0, CH // L, step=1, unroll=2)
            def _cgrp(q):
                iv = ib[pl.ds(q * L, L)]
                for tt in range(L):
                    a = iv[tt]
                    plsc.addupdate(cnt_acc.at[a, pl.ds(0, L)], ones16)

    _start_in(0, kbufa, vbufa, idxa, sina)

    def _pair(p, _):
        _wait_in(kbufa, vbufa, idxa, sina)
        _start_in(2 * p + 1, kbufb, vbufb, idxb, sinb)
        _compute(kbufa, vbufa, idxa)
        _wait_in(kbufb, vbufb, idxb, sinb)

        @pl.when(p < NCH // 2 - 1)
        def _():
            _start_in(2 * p + 2, kbufa, vbufa, idxa, sina)

        _compute(kbufb, vbufb, idxb)
        return 0

    lax.fori_loop(0, NCH // 2, _pair, 0)

    pltpu.sync_copy(acc_k, stage_k.at[sid])
    pltpu.sync_copy(acc_v, stage_v.at[sid])

    @pl.when(gl == 0)
    def _():
        pltpu.sync_copy(cnt_acc, stage_c.at[t])

    plsc.subcore_barrier()

    pltpu.sync_copy(stage_k.at[gl * NT].at[pl.ds(r0, RPT)], skbuf)
    pltpu.sync_copy(stage_v.at[gl * NT].at[pl.ds(r0, RPT)], svbuf)
    pltpu.sync_copy(stage_c.at[0].at[pl.ds(r0, RPT)], mr)
    for t2 in range(1, NT):
        pltpu.sync_copy(stage_k.at[gl * NT + t2].at[pl.ds(r0, RPT)], tbuf)

        def _addk(r, _):
            for j in range(DG // L):
                sl = pl.ds(j * L, L)
                skbuf[r, sl] = skbuf[r, sl] + tbuf[r, sl]
            return 0
        lax.fori_loop(0, RPT, _addk, 0)
        pltpu.sync_copy(stage_v.at[gl * NT + t2].at[pl.ds(r0, RPT)], tbuf)

        def _addv(r, _):
            for j in range(DG // L):
                sl = pl.ds(j * L, L)
                svbuf[r, sl] = svbuf[r, sl] + tbuf[r, sl]
            return 0
        lax.fori_loop(0, RPT, _addv, 0)
        pltpu.sync_copy(stage_c.at[t2].at[pl.ds(r0, RPT)], ctmp)

        def _addc(r, _):
            sl = pl.ds(0, L)
            mr[r, sl] = mr[r, sl] + ctmp[r, sl]
            return 0
        lax.fori_loop(0, RPT, _addc, 0)

    pltpu.sync_copy(cnt0_hbm.at[pl.ds(r0, RPT)], nr)
    pltpu.sync_copy(cent_hbm.at[pl.ds(r0, RPT), pl.ds(gcol, DG)], cbuf)
    pltpu.sync_copy(st_hbm.at[pl.ds(r0, RPT), pl.ds(gcol, DG)], sbuf)

    def _mg(r, _):
        nvec = nr[r, pl.ds(0, L)]
        mvec = mr[r, pl.ds(0, L)]
        denom = nvec + mvec
        pos = denom > 0.5
        inv = 1.0 / jnp.where(pos, denom, 1.0)
        for j in range(DG // L):
            sl = pl.ds(j * L, L)
            c = cbuf[r, sl]
            s = sbuf[r, sl]
            cbuf[r, sl] = jnp.where(pos, (nvec * c + skbuf[r, sl]) * inv, c)
            sbuf[r, sl] = jnp.where(pos, (nvec * s + svbuf[r, sl]) * inv, s)
        return 0

    lax.fori_loop(0, RPT, _mg, 0)
    pltpu.sync_copy(cbuf, outc_hbm.at[pl.ds(r0, RPT), pl.ds(gcol, DG)])
    pltpu.sync_copy(sbuf, outs_hbm.at[pl.ds(r0, RPT), pl.ds(gcol, DG)])

    @pl.when((cid == 0) & (gl == 0))
    def _():
        lane = lax.iota(jnp.int32, L)
        acc = zf16
        for r in range(RPT):
            row = mr[r, pl.ds(0, L)] + nr[r, pl.ds(0, L)]
            acc = jnp.where(lane == r, row, acc)
        outcnt[:] = acc.astype(jnp.int32)
        pltpu.sync_copy(outcnt, outn_hbm.at[pl.ds(r0, RPT)])


@jax.jit
def _run(centroids, states, keys, values, assign, cnt0):
    mesh = plsc.VectorSubcoreMesh(core_axis_name="c", subcore_axis_name="s")
    f = pl.kernel(
        _body,
        out_type=(
            jax.ShapeDtypeStruct((K, D), jnp.float32),
            jax.ShapeDtypeStruct((K, D), jnp.float32),
            jax.ShapeDtypeStruct((K,), jnp.int32),
        ),
        mesh=mesh,
        scratch_types=[
            pltpu.VMEM_SHARED((NS, K, DG), jnp.float32),
            pltpu.VMEM_SHARED((NS, K, DG), jnp.float32),
            pltpu.VMEM_SHARED((NT, K, DG), jnp.float32),
            pltpu.VMEM((K, DG), jnp.float32),
            pltpu.VMEM((K, DG), jnp.float32),
            pltpu.VMEM((K, DG), jnp.float32),
            pltpu.VMEM((CH, DG), jnp.float32),
            pltpu.VMEM((CH, DG), jnp.float32),
            pltpu.VMEM((CH,), jnp.int32),
            pltpu.VMEM((CH, DG), jnp.float32),
            pltpu.VMEM((CH, DG), jnp.float32),
            pltpu.VMEM((CH,), jnp.int32),
            pltpu.VMEM((RPT, DG), jnp.float32),
            pltpu.VMEM((RPT, DG), jnp.float32),
            pltpu.VMEM((RPT, DG), jnp.float32),
            pltpu.VMEM((RPT, DG), jnp.float32),
            pltpu.VMEM((RPT, DG), jnp.float32),
            pltpu.VMEM((RPT, DG), jnp.float32),
            pltpu.VMEM((RPT, DG), jnp.float32),
            pltpu.VMEM((RPT, DG), jnp.float32),
            pltpu.VMEM((RPT,), jnp.int32),
            pltpu.SemaphoreType.DMA,
            pltpu.SemaphoreType.DMA,
        ],
    )
    return f(centroids, states, keys, values, assign, cnt0)


def kernel(centroids, states, counts, keys, values, assign):
    assign = assign.astype(jnp.int32)
    cnt0 = jnp.broadcast_to(
        counts.astype(jnp.float32)[:, None], (K, DG))
    return _run(centroids, states, keys, values, assign, cnt0)


# speedup vs baseline: 1.1849x; 1.1849x over previous
import jax
import jax.numpy as jnp
from jax import lax
from jax.experimental import pallas as pl
from jax.experimental.pallas import tpu as pltpu
from jax.experimental.pallas import tpu_sc as plsc

K = 64
D = 1024
N = 8192
NC = 2
NS = 16
L = 16
NG = 4
NT = 4
DG = 128
DH = NG * DG
TOK_TC = 3072
TOK_SC = N - TOK_TC
TPG = TOK_SC // NT
RPT = K // NT
CH = 128
NCH = TPG // CH
TB = 512
GB = TOK_TC // TB


def _sc_body(keys_hbm, vals_hbm, asg_hbm,
             outk_hbm, outv_hbm, outm_hbm,
             stage_k, stage_v, stage_c,
             acc_k, acc_v, cnt_acc,
             kbufa, vbufa, idxa, kbufb, vbufb, idxb,
             skbuf, svbuf, tbuf, mr, ctmp,
             sina, sinb):
    cid = lax.axis_index("c")
    sid = lax.axis_index("s")
    gl = sid // NT
    t = sid % NT
    gcol = cid * DH + gl * DG
    tok0 = TOK_TC + t * TPG
    r0 = t * RPT

    zf16 = jnp.zeros((L,), jnp.float32)
    ones16 = jnp.ones((L,), jnp.float32)

    def _zf(r, _):
        for j in range(DG // L):
            acc_k[r, pl.ds(j * L, L)] = zf16
            acc_v[r, pl.ds(j * L, L)] = zf16
        cnt_acc[r, pl.ds(0, L)] = zf16
        return 0
    lax.fori_loop(0, K, _zf, 0)

    def _start_in(c, kb, vb, ib, sem):
        base = tok0 + c * CH
        pltpu.async_copy(asg_hbm.at[pl.ds(base, CH)], ib, sem)
        pltpu.async_copy(
            keys_hbm.at[pl.ds(base, CH), pl.ds(gcol, DG)], kb, sem)
        pltpu.async_copy(
            vals_hbm.at[pl.ds(base, CH), pl.ds(gcol, DG)], vb, sem)

    def _wait_in(kb, vb, ib, sem):
        pltpu.make_async_copy(asg_hbm.at[pl.ds(0, CH)], ib, sem).wait()
        pltpu.make_async_copy(
            keys_hbm.at[pl.ds(0, CH), pl.ds(0, DG)], kb, sem).wait()
        pltpu.make_async_copy(
            vals_hbm.at[pl.ds(0, CH), pl.ds(0, DG)], vb, sem).wait()

    def _compute(kb, vb, ib):
        @plsc.parallel_loop(0, CH // L, step=1, unroll=2)
        def _grp(q):
            iv = ib[pl.ds(q * L, L)]

            def _loads(tok):
                return (
                    [kb[tok, pl.ds(j * L, L)] for j in range(DG // L)],
                    [vb[tok, pl.ds(j * L, L)] for j in range(DG // L)],
                )

            cur = _loads(q * L)
            acur = iv[0]
            for tt in range(L):
                if tt + 1 < L:
                    nxt = _loads(q * L + tt + 1)
                    anxt = iv[tt + 1]
                kcur, vcur = cur
                for j in range(DG // L):
                    plsc.addupdate(acc_k.at[acur, pl.ds(j * L, L)], kcur[j])
                for j in range(DG // L):
                    plsc.addupdate(acc_v.at[acur, pl.ds(j * L, L)], vcur[j])
                if tt + 1 < L:
                    cur = nxt
                    acur = anxt

        @pl.when(gl == 0)
        def _():
            @plsc.parallel_loop(0, CH // L, step=1, unroll=2)
            def _cgrp(q):
                iv = ib[pl.ds(q * L, L)]
                for tt in range(L):
                    a = iv[tt]
                    plsc.addupdate(cnt_acc.at[a, pl.ds(0, L)], ones16)

    _start_in(0, kbufa, vbufa, idxa, sina)

    def _pair(p, _):
        _wait_in(kbufa, vbufa, idxa, sina)
        _start_in(2 * p + 1, kbufb, vbufb, idxb, sinb)
        _compute(kbufa, vbufa, idxa)
        _wait_in(kbufb, vbufb, idxb, sinb)

        @pl.when(p < NCH // 2 - 1)
        def _():
            _start_in(2 * p + 2, kbufa, vbufa, idxa, sina)

        _compute(kbufb, vbufb, idxb)
        return 0

    lax.fori_loop(0, NCH // 2, _pair, 0)

    pltpu.sync_copy(acc_k, stage_k.at[sid])
    pltpu.sync_copy(acc_v, stage_v.at[sid])

    @pl.when(gl == 0)
    def _():
        pltpu.sync_copy(cnt_acc, stage_c.at[t])

    plsc.subcore_barrier()

    pltpu.sync_copy(stage_k.at[gl * NT].at[pl.ds(r0, RPT)], skbuf)
    pltpu.sync_copy(stage_v.at[gl * NT].at[pl.ds(r0, RPT)], svbuf)
    pltpu.sync_copy(stage_c.at[0].at[pl.ds(r0, RPT)], mr)
    for t2 in range(1, NT):
        pltpu.sync_copy(stage_k.at[gl * NT + t2].at[pl.ds(r0, RPT)], tbuf)

        def _addk(r, _):
            for j in range(DG // L):
                sl = pl.ds(j * L, L)
                skbuf[r, sl] = skbuf[r, sl] + tbuf[r, sl]
            return 0
        lax.fori_loop(0, RPT, _addk, 0)
        pltpu.sync_copy(stage_v.at[gl * NT + t2].at[pl.ds(r0, RPT)], tbuf)

        def _addv(r, _):
            for j in range(DG // L):
                sl = pl.ds(j * L, L)
                svbuf[r, sl] = svbuf[r, sl] + tbuf[r, sl]
            return 0
        lax.fori_loop(0, RPT, _addv, 0)
        pltpu.sync_copy(stage_c.at[t2].at[pl.ds(r0, RPT)], ctmp)

        def _addc(r, _):
            sl = pl.ds(0, L)
            mr[r, sl] = mr[r, sl] + ctmp[r, sl]
            return 0
        lax.fori_loop(0, RPT, _addc, 0)

    pltpu.sync_copy(skbuf, outk_hbm.at[pl.ds(r0, RPT), pl.ds(gcol, DG)])
    pltpu.sync_copy(svbuf, outv_hbm.at[pl.ds(r0, RPT), pl.ds(gcol, DG)])

    @pl.when((cid == 0) & (gl == 0))
    def _():
        pltpu.sync_copy(mr, outm_hbm.at[pl.ds(r0, RPT)])


def _sc_sum(keys, values, assign):
    mesh = plsc.VectorSubcoreMesh(core_axis_name="c", subcore_axis_name="s")
    f = pl.kernel(
        _sc_body,
        out_type=(
            jax.ShapeDtypeStruct((K, D), jnp.float32),
            jax.ShapeDtypeStruct((K, D), jnp.float32),
            jax.ShapeDtypeStruct((K, DG), jnp.float32),
        ),
        mesh=mesh,
        scratch_types=[
            pltpu.VMEM_SHARED((NS, K, DG), jnp.float32),
            pltpu.VMEM_SHARED((NS, K, DG), jnp.float32),
            pltpu.VMEM_SHARED((NT, K, DG), jnp.float32),
            pltpu.VMEM((K, DG), jnp.float32),
            pltpu.VMEM((K, DG), jnp.float32),
            pltpu.VMEM((K, DG), jnp.float32),
            pltpu.VMEM((CH, DG), jnp.float32),
            pltpu.VMEM((CH, DG), jnp.float32),
            pltpu.VMEM((CH,), jnp.int32),
            pltpu.VMEM((CH, DG), jnp.float32),
            pltpu.VMEM((CH, DG), jnp.float32),
            pltpu.VMEM((CH,), jnp.int32),
            pltpu.VMEM((RPT, DG), jnp.float32),
            pltpu.VMEM((RPT, DG), jnp.float32),
            pltpu.VMEM((RPT, DG), jnp.float32),
            pltpu.VMEM((RPT, DG), jnp.float32),
            pltpu.VMEM((RPT, DG), jnp.float32),
            pltpu.SemaphoreType.DMA,
            pltpu.SemaphoreType.DMA,
        ],
    )
    return f(keys, values, assign)


def _tc_sum_body(asg_ref, k_ref, v_ref, ok_ref, ov_ref, oc_ref):
    b = pl.program_id(0)
    a = asg_ref[0, 0, :]
    oh = (lax.broadcasted_iota(jnp.int32, (K, TB), 0)
          == a[None, :]).astype(jnp.float32)
    ks = jnp.dot(oh, k_ref[...], preferred_element_type=jnp.float32)
    vs = jnp.dot(oh, v_ref[...], preferred_element_type=jnp.float32)
    cs = jnp.broadcast_to(jnp.sum(oh, axis=1, keepdims=True), (K, DG))

    @pl.when(b == 0)
    def _():
        ok_ref[...] = ks
        ov_ref[...] = vs
        oc_ref[...] = cs

    @pl.when(b > 0)
    def _():
        ok_ref[...] = ok_ref[...] + ks
        ov_ref[...] = ov_ref[...] + vs
        oc_ref[...] = oc_ref[...] + cs


def _tc_sum(keys, values, asg3):
    return pl.pallas_call(
        _tc_sum_body,
        grid=(GB,),
        in_specs=[
            pl.BlockSpec((1, 1, TB), lambda b: (b, 0, 0)),
            pl.BlockSpec((TB, D), lambda b: (b, 0)),
            pl.BlockSpec((TB, D), lambda b: (b, 0)),
        ],
        out_specs=[
            pl.BlockSpec((K, D), lambda b: (0, 0)),
            pl.BlockSpec((K, D), lambda b: (0, 0)),
            pl.BlockSpec((K, DG), lambda b: (0, 0)),
        ],
        out_shape=[
            jax.ShapeDtypeStruct((K, D), jnp.float32),
            jax.ShapeDtypeStruct((K, D), jnp.float32),
            jax.ShapeDtypeStruct((K, DG), jnp.float32),
        ],
    )(asg3, keys[:TOK_TC], values[:TOK_TC])


def _merge_body(cent_ref, st_ref, cnt0_ref, sk_ref, sv_ref, tk_ref, tv_ref,
                ms_ref, mt_ref, oc_ref, os_ref, on_ref):
    dn128 = cnt0_ref[...] + ms_ref[...] + mt_ref[...]
    n = cnt0_ref[:, :1]
    denom = dn128[:, :1]
    pos = denom > 0.5
    inv = 1.0 / jnp.where(pos, denom, 1.0)
    cent = cent_ref[...]
    st = st_ref[...]
    oc_ref[...] = jnp.where(
        pos, (n * cent + sk_ref[...] + tk_ref[...]) * inv, cent)
    os_ref[...] = jnp.where(
        pos, (n * st + sv_ref[...] + tv_ref[...]) * inv, st)
    on_ref[...] = dn128


def _merge(centroids, states, cnt0, sk, sv, tk, tv, ms, mt):
    return pl.pallas_call(
        _merge_body,
        out_shape=[
            jax.ShapeDtypeStruct((K, D), jnp.float32),
            jax.ShapeDtypeStruct((K, D), jnp.float32),
            jax.ShapeDtypeStruct((K, DG), jnp.float32),
        ],
    )(centroids, states, cnt0, sk, sv, tk, tv, ms, mt)


@jax.jit
def _run(centroids, states, counts, keys, values, assign):
    cnt0 = jnp.broadcast_to(
        counts.astype(jnp.float32)[:, None], (K, DG))
    asg3 = assign[:TOK_TC].reshape(GB, 1, TB)
    sk, sv, ms = _sc_sum(keys, values, assign)
    tk, tv, mt = _tc_sum(keys, values, asg3)
    oc, os, dn = _merge(centroids, states, cnt0, sk, sv, tk, tv, ms, mt)
    new_counts = dn[:, 0].astype(counts.dtype)
    return oc, os, new_counts


def kernel(centroids, states, counts, keys, values, assign):
    assign = assign.astype(jnp.int32)
    counts = counts.astype(jnp.int32)
    return _run(centroids, states, counts, keys, values, assign)
